# baseline (speedup 1.0000x reference)
import functools

import jax
import jax.numpy as jnp
from jax import lax
from jax.experimental import pallas as pl
from jax.experimental.pallas import tpu as pltpu
from jax.experimental.pallas import tpu_sc as plsc

VOCAB = 100000
HIDDEN = 1024
BATCH = 4
SEQ = 4096

NC = 2
NS = 16
NW = NC * NS

B = BATCH * SEQ
B_PER_W = B // NW
CHUNK = 16
N_CHUNKS = B_PER_W // CHUNK
NBUF = 3
SBUF = 3
W_PER_ROW = SEQ // B_PER_W


@functools.partial(
    pl.kernel,
    out_type=jax.ShapeDtypeStruct((B, HIDDEN), jnp.float32),
    mesh=plsc.VectorSubcoreMesh(core_axis_name="c", subcore_axis_name="s"),
    scratch_types=[
        pltpu.VMEM((B_PER_W,), jnp.int32),
        pltpu.VMEM((NBUF, CHUNK, HIDDEN), jnp.float32),
        pltpu.VMEM_SHARED((NS, SBUF, CHUNK, HIDDEN), jnp.float32),
        pltpu.SemaphoreType.DMA((NBUF,)),
        pltpu.SemaphoreType.DMA((SBUF,)),
        pltpu.SemaphoreType.DMA((SBUF,)),
    ],
)
def _embed_sc(ids_hbm, tab_hbm, out_hbm, idx_v, buf, spm_all, gsem, xsem, psem):
    wid = lax.axis_index("s") * NC + lax.axis_index("c")
    spm = spm_all.at[lax.axis_index("s")]
    chunk0 = wid * N_CHUNKS
    pltpu.sync_copy(
        ids_hbm.at[wid // W_PER_ROW,
                   pl.ds((wid % W_PER_ROW) * B_PER_W, B_PER_W)],
        idx_v,
    )

    def G(g):
        pltpu.async_copy(
            tab_hbm.at[idx_v.at[pl.ds(g * CHUNK, CHUNK)]],
            buf.at[g % NBUF], gsem.at[g % NBUF],
        )

    def wait_G(g):
        pltpu.make_async_copy(
            tab_hbm.at[idx_v.at[pl.ds(g * CHUNK, CHUNK)]],
            buf.at[g % NBUF], gsem.at[g % NBUF],
        ).wait()

    def X(g):
        pltpu.async_copy(buf.at[g % NBUF], spm.at[g % SBUF], xsem.at[g % SBUF])

    def wait_X(g):
        pltpu.make_async_copy(
            buf.at[g % NBUF], spm.at[g % SBUF], xsem.at[g % SBUF]
        ).wait()

    def P(g):
        pltpu.async_copy(
            spm.at[g % SBUF],
            out_hbm.at[pl.ds((chunk0 + g) * CHUNK, CHUNK)],
            psem.at[g % SBUF],
        )

    def wait_P(g):
        pltpu.make_async_copy(
            spm.at[g % SBUF],
            out_hbm.at[pl.ds((chunk0 + g) * CHUNK, CHUNK)],
            psem.at[g % SBUF],
        ).wait()

    G(0)
    G(1)
    for g in range(N_CHUNKS):
        wait_G(g)
        if g >= SBUF:
            wait_P(g - SBUF)
        X(g)
        if g >= 1:
            wait_X(g - 1)
            P(g - 1)
            if g + 1 < N_CHUNKS:
                G(g + 1)
    wait_X(N_CHUNKS - 1)
    P(N_CHUNKS - 1)
    for g in range(N_CHUNKS - SBUF, N_CHUNKS):
        wait_P(g)


def kernel(input_ids, word_embeddings):
    out = _embed_sc(input_ids.astype(jnp.int32), word_embeddings)
    return out.reshape(BATCH, SEQ, HIDDEN)


# speedup vs baseline: 1.1155x; 1.1155x over previous
import functools

import jax
import jax.numpy as jnp
from jax import lax
from jax.experimental import pallas as pl
from jax.experimental.pallas import tpu as pltpu
from jax.experimental.pallas import tpu_sc as plsc

VOCAB = 100000
HIDDEN = 1024
BATCH = 4
SEQ = 4096

NC = 2
NS = 16
NW = NC * NS

B = BATCH * SEQ
B_PER_W = B // NW
CHUNK = 32
N_CHUNKS = B_PER_W // CHUNK
NBUF = 2
W_PER_ROW = SEQ // B_PER_W


@functools.partial(
    pl.kernel,
    out_type=jax.ShapeDtypeStruct((B, HIDDEN), jnp.float32),
    mesh=plsc.VectorSubcoreMesh(core_axis_name="c", subcore_axis_name="s"),
    scratch_types=[
        pltpu.VMEM((B_PER_W,), jnp.int32),
        pltpu.VMEM((NBUF, CHUNK, HIDDEN), jnp.float32),
        pltpu.SemaphoreType.DMA((NBUF,)),
        pltpu.SemaphoreType.DMA((NBUF,)),
    ],
)
def _embed_sc(ids_hbm, tab_hbm, out_hbm, idx_v, buf, gsem, osem):
    wid = lax.axis_index("s") * NC + lax.axis_index("c")
    chunk0 = wid * N_CHUNKS
    pltpu.sync_copy(
        ids_hbm.at[wid // W_PER_ROW,
                   pl.ds((wid % W_PER_ROW) * B_PER_W, B_PER_W)],
        idx_v,
    )

    sizes = [16] + [32] * ((B_PER_W - 32) // 32) + [16]
    offs = [0]
    for sz in sizes[:-1]:
        offs.append(offs[-1] + sz)
    n = len(sizes)

    def gather(g):
        pltpu.async_copy(
            tab_hbm.at[idx_v.at[pl.ds(offs[g], sizes[g])]],
            buf.at[g % NBUF].at[pl.ds(0, sizes[g])], gsem.at[g % NBUF],
        )

    def wait_gather(g):
        pltpu.make_async_copy(
            tab_hbm.at[idx_v.at[pl.ds(offs[g], sizes[g])]],
            buf.at[g % NBUF].at[pl.ds(0, sizes[g])], gsem.at[g % NBUF],
        ).wait()

    def put(g):
        pltpu.async_copy(
            buf.at[g % NBUF].at[pl.ds(0, sizes[g])],
            out_hbm.at[pl.ds(wid * B_PER_W + offs[g], sizes[g])],
            osem.at[g % NBUF],
        )

    def wait_put(g):
        pltpu.make_async_copy(
            buf.at[g % NBUF].at[pl.ds(0, sizes[g])],
            out_hbm.at[pl.ds(wid * B_PER_W + offs[g], sizes[g])],
            osem.at[g % NBUF],
        ).wait()

    gather(0)
    for g in range(n):
        nxt = g + 1
        if nxt < n:
            if nxt >= NBUF:
                wait_put(nxt - NBUF)
            gather(nxt)
        wait_gather(g)
        put(g)
    for g in range(n - NBUF, n):
        wait_put(g)


def kernel(input_ids, word_embeddings):
    out = _embed_sc(input_ids.astype(jnp.int32), word_embeddings)
    return out.reshape(BATCH, SEQ, HIDDEN)


# speedup vs baseline: 1.1327x; 1.0155x over previous
import functools

import jax
import jax.numpy as jnp
from jax import lax
from jax.experimental import pallas as pl
from jax.experimental.pallas import tpu as pltpu
from jax.experimental.pallas import tpu_sc as plsc

VOCAB = 100000
HIDDEN = 1024
BATCH = 4
SEQ = 4096

NC = 2
NS = 16
NW = NC * NS

B = BATCH * SEQ
B_PER_W = B // NW
CHUNK = 32
N_CHUNKS = B_PER_W // CHUNK
NBUF = 2
W_PER_ROW = SEQ // B_PER_W


@functools.partial(
    pl.kernel,
    out_type=jax.ShapeDtypeStruct((B, HIDDEN), jnp.float32),
    mesh=plsc.VectorSubcoreMesh(core_axis_name="c", subcore_axis_name="s"),
    scratch_types=[
        pltpu.VMEM((B_PER_W,), jnp.int32),
        pltpu.VMEM((NBUF, CHUNK, HIDDEN), jnp.float32),
        pltpu.SemaphoreType.DMA((NBUF,)),
        pltpu.SemaphoreType.DMA((NBUF,)),
    ],
)
def _embed_sc(ids_hbm, tab_hbm, out_hbm, idx_v, buf, gsem, osem):
    wid = lax.axis_index("s") * NC + lax.axis_index("c")
    chunk0 = wid * N_CHUNKS
    pltpu.sync_copy(
        ids_hbm.at[wid // W_PER_ROW,
                   pl.ds((wid % W_PER_ROW) * B_PER_W, B_PER_W)],
        idx_v,
    )

    def gather(g, b):
        pltpu.async_copy(tab_hbm.at[idx_v.at[pl.ds(g * CHUNK, CHUNK)]], buf.at[b], gsem.at[b])

    def wait_gather(g, b):
        pltpu.make_async_copy(
            tab_hbm.at[idx_v.at[pl.ds(g * CHUNK, CHUNK)]], buf.at[b], gsem.at[b]
        ).wait()

    def put(g, b):
        pltpu.async_copy(
            buf.at[b], out_hbm.at[pl.ds((chunk0 + g) * CHUNK, CHUNK)],
            osem.at[b],
        )

    def wait_put(g, b):
        pltpu.make_async_copy(
            buf.at[b], out_hbm.at[pl.ds((chunk0 + g) * CHUNK, CHUNK)],
            osem.at[b],
        ).wait()

    gather(0, 0)
    gather(1, 1)
    wait_gather(0, 0)
    put(0, 0)

    def steady(o, _):
        for s in range(NBUF):
            g = 1 + o * NBUF + s
            b = (1 + s) % NBUF
            bn = s % NBUF
            wait_put(g - 1, bn)
            gather(g + 1, bn)
            wait_gather(g, b)
            put(g, b)
        return _

    lax.fori_loop(0, (N_CHUNKS - 2) // NBUF, steady, None)

    g = N_CHUNKS - 1
    wait_gather(g, g % NBUF)
    put(g, g % NBUF)
    wait_put(g - 1, (g - 1) % NBUF)
    wait_put(g, g % NBUF)


def kernel(input_ids, word_embeddings):
    out = _embed_sc(input_ids.astype(jnp.int32), word_embeddings)
    return out.reshape(BATCH, SEQ, HIDDEN)
